# baseline (speedup 1.0000x reference)
import jax
import jax.numpy as jnp
from jax.experimental import pallas as pl
from jax.experimental.pallas import tpu as pltpu

_BB = 16
_CH = 8192
_NCH = 13


def _mm_kernel(params_ref, x_ref, tx_ref, mem_ref, out_t_ref, out_ref, memt):
    i = pl.program_id(0)
    inv_t = 1.0 / params_ref[0]

    @pl.when(i < _NCH)
    def _transpose_chunk():
        ra = jax.lax.broadcasted_iota(jnp.int32, (64, 64), 0)
        rb = jax.lax.broadcasted_iota(jnp.int32, (64, 64), 1)
        eye = (ra == rb).astype(jnp.float32)
        memt[i] = jax.lax.dot_general(
            eye, mem_ref[...], (((1,), (1,)), ((), ())),
            preferred_element_type=jnp.float32)

    @pl.when(i >= _NCH)
    def _compute():
        xx = jnp.concatenate([x_ref[...], tx_ref[...]], axis=0) * inv_t
        M = out_ref.shape[1]
        for j in range(_NCH):
            y = jax.lax.dot_general(
                xx, memt[j], (((1,), (0,)), ((), ())),
                preferred_element_type=jnp.float32)
            w = min(_CH, M - j * _CH)
            out_ref[:, j * _CH:j * _CH + w] = y[:_BB, :w]
            out_t_ref[:, j * _CH:j * _CH + w] = y[_BB:, :w]


@jax.jit
def kernel(image_features, transformed_image_features, indices, memory, params):
    del indices
    B, D = image_features.shape
    M = memory.shape[0]
    grid = (_NCH + B // _BB,)
    out_shape = jax.ShapeDtypeStruct((B, M), jnp.float32)
    out_t, out = pl.pallas_call(
        _mm_kernel,
        grid=grid,
        in_specs=[
            pl.BlockSpec(memory_space=pltpu.SMEM),
            pl.BlockSpec((_BB, D), lambda i: (jnp.maximum(i - _NCH, 0), 0)),
            pl.BlockSpec((_BB, D), lambda i: (jnp.maximum(i - _NCH, 0), 0)),
            pl.BlockSpec((_CH, D), lambda i: (jnp.minimum(i, _NCH - 1), 0)),
        ],
        out_specs=[
            pl.BlockSpec((_BB, M), lambda i: (jnp.maximum(i - _NCH, 0), 0)),
            pl.BlockSpec((_BB, M), lambda i: (jnp.maximum(i - _NCH, 0), 0)),
        ],
        out_shape=[out_shape, out_shape],
        scratch_shapes=[
            pltpu.VMEM((_NCH, 64, _CH), jnp.float32),
        ],
        compiler_params=pltpu.CompilerParams(
            dimension_semantics=("arbitrary",),
        ),
    )(params, image_features, transformed_image_features, memory)
    return (out_t, out)


# speedup vs baseline: 3.9698x; 3.9698x over previous
import jax
import jax.numpy as jnp
from jax.experimental import pallas as pl
from jax.experimental.pallas import tpu as pltpu

_BM = 2048


def _mm_kernel(params_ref, xt_ref, txt_ref, memt_ref, out_t_ref, out_ref):
    inv_t = 1.0 / params_ref[0]
    m = memt_ref[...]
    xs = xt_ref[...] * inv_t
    txs = txt_ref[...] * inv_t
    dn = (((0,), (0,)), ((), ()))
    out_ref[...] = jax.lax.dot_general(
        m, xs, dn, preferred_element_type=jnp.float32)
    out_t_ref[...] = jax.lax.dot_general(
        m, txs, dn, preferred_element_type=jnp.float32)


@jax.jit
def kernel(image_features, transformed_image_features, indices, memory, params):
    del indices
    B, D = image_features.shape
    M = memory.shape[0]
    xt = image_features.T
    txt = transformed_image_features.T
    memt = memory.T
    grid = (pl.cdiv(M, _BM),)
    out_shape = jax.ShapeDtypeStruct((M, B), jnp.float32)
    out_t_p, out_p = pl.pallas_call(
        _mm_kernel,
        grid=grid,
        in_specs=[
            pl.BlockSpec(memory_space=pltpu.SMEM),
            pl.BlockSpec((D, B), lambda j: (0, 0)),
            pl.BlockSpec((D, B), lambda j: (0, 0)),
            pl.BlockSpec((D, _BM), lambda j: (0, j)),
        ],
        out_specs=[
            pl.BlockSpec((_BM, B), lambda j: (j, 0)),
            pl.BlockSpec((_BM, B), lambda j: (j, 0)),
        ],
        out_shape=[out_shape, out_shape],
        compiler_params=pltpu.CompilerParams(
            dimension_semantics=("arbitrary",),
        ),
    )(params, xt, txt, memt)
    return (out_t_p.T, out_p.T)
